# Initial kernel scaffold; baseline (speedup 1.0000x reference)
import functools

import jax
import jax.numpy as jnp
from jax import lax
from jax.experimental import pallas as pl
from jax.experimental.pallas import tpu as pltpu
from jax.experimental.pallas import tpu_sc as plsc

N_NODES = 10000
N_EDGES = 320000
D = 128

NC = 2
NS = 16
NW = NC * NS

CHUNK = 128
E_PER_W = N_EDGES // NW
CH_MAIN = E_PER_W // CHUNK
TAIL = E_PER_W - CH_MAIN * CHUNK
ROWS_PER_TILE = N_NODES // NS


def _sc_gather_scatter(feature, src_main, dst_main, src_tail, dst_tail, zeros):
  mesh = plsc.VectorSubcoreMesh(core_axis_name="c", subcore_axis_name="s")

  @functools.partial(
      pl.kernel,
      mesh=mesh,
      out_type=jax.ShapeDtypeStruct((NC * N_NODES, D), jnp.float32),
      scratch_types=[
          pltpu.VMEM((CH_MAIN, CHUNK), jnp.int32),
          pltpu.VMEM((CH_MAIN, CHUNK), jnp.int32),
          pltpu.VMEM((1, TAIL), jnp.int32),
          pltpu.VMEM((1, TAIL), jnp.int32),
          pltpu.VMEM((CHUNK, D), jnp.float32),
          pltpu.VMEM((TAIL, D), jnp.float32),
          pltpu.VMEM_SHARED((N_NODES, D), jnp.float32),
          pltpu.SemaphoreType.DMA,
      ],
  )
  def k(feat_hbm, srcm_hbm, dstm_hbm, srct_hbm, dstt_hbm, zero_hbm, out_hbm,
        srcm_v, dstm_v, srct_v, dstt_v, rows_v, tail_v, acc_s, sem):
    c = lax.axis_index("c")
    s = lax.axis_index("s")
    wid = s * NC + c

    pltpu.sync_copy(zero_hbm,
                    acc_s.at[pl.ds(s * ROWS_PER_TILE, ROWS_PER_TILE)])
    pltpu.sync_copy(srcm_hbm.at[wid], srcm_v)
    pltpu.sync_copy(dstm_hbm.at[wid], dstm_v)
    pltpu.sync_copy(srct_hbm.at[wid], srct_v)
    pltpu.sync_copy(dstt_hbm.at[wid], dstt_v)
    plsc.subcore_barrier()

    def body(j, carry):
      pltpu.async_copy(feat_hbm.at[srcm_v.at[j]], rows_v, sem).wait()
      pltpu.sync_copy(rows_v, acc_s.at[dstm_v.at[j]], add=True)
      return carry

    lax.fori_loop(0, CH_MAIN, body, 0)

    pltpu.async_copy(feat_hbm.at[srct_v.at[0]], tail_v, sem).wait()
    pltpu.sync_copy(tail_v, acc_s.at[dstt_v.at[0]], add=True)

    plsc.subcore_barrier()
    pltpu.sync_copy(
        acc_s.at[pl.ds(s * ROWS_PER_TILE, ROWS_PER_TILE)],
        out_hbm.at[pl.ds(c * N_NODES + s * ROWS_PER_TILE, ROWS_PER_TILE)])

  return k(feature, src_main, dst_main, src_tail, dst_tail, zeros)


def _tc_linear_kernel(h0_ref, h1_ref, w_ref, b_ref, o_ref):
  h = h0_ref[...] + h1_ref[...]
  o_ref[...] = lax.dot_general(
      h, w_ref[...], (((1,), (1,)), ((), ())),
      preferred_element_type=jnp.float32) + b_ref[...]


def kernel(feature, edge_index, W, b):
  src = edge_index[0]
  dst = edge_index[1]
  n_main = NW * CH_MAIN * CHUNK
  src_main = src[:n_main].reshape(NW, CH_MAIN, CHUNK)
  dst_main = dst[:n_main].reshape(NW, CH_MAIN, CHUNK)
  src_tail = src[n_main:].reshape(NW, 1, TAIL)
  dst_tail = dst[n_main:].reshape(NW, 1, TAIL)
  zeros = jnp.zeros((ROWS_PER_TILE, D), jnp.float32)

  partials = _sc_gather_scatter(feature, src_main, dst_main,
                                src_tail, dst_tail, zeros)

  out = pl.pallas_call(
      _tc_linear_kernel,
      grid=(1,),
      in_specs=[
          pl.BlockSpec((N_NODES, D), lambda i: (0, 0)),
          pl.BlockSpec((N_NODES, D), lambda i: (1, 0)),
          pl.BlockSpec((D, D), lambda i: (0, 0)),
          pl.BlockSpec((1, D), lambda i: (0, 0)),
      ],
      out_specs=pl.BlockSpec((N_NODES, D), lambda i: (0, 0)),
      out_shape=jax.ShapeDtypeStruct((N_NODES, D), jnp.float32),
  )(partials, partials, W, b.reshape(1, D))
  return out


# speedup vs baseline: 8.3935x; 8.3935x over previous
import functools

import jax
import jax.numpy as jnp
from jax import lax
from jax.experimental import pallas as pl
from jax.experimental.pallas import tpu as pltpu
from jax.experimental.pallas import tpu_sc as plsc

N_NODES = 10000
N_EDGES = 320000
D = 128

NC = 2
NS = 16
NW = NC * NS

CHUNK = 128
E_PER_W = N_EDGES // NW
CH_MAIN = E_PER_W // CHUNK
TAIL = E_PER_W - CH_MAIN * CHUNK
ROWS_PER_TILE = 624
ROWS_REM = N_NODES - NS * ROWS_PER_TILE


def _sc_gather_scatter(feature, src_main, dst_main, src_tail, dst_tail, zeros):
  mesh = plsc.VectorSubcoreMesh(core_axis_name="c", subcore_axis_name="s")

  @functools.partial(
      pl.kernel,
      mesh=mesh,
      out_type=jax.ShapeDtypeStruct((NC * N_NODES, D), jnp.float32),
      scratch_types=[
          pltpu.VMEM((CH_MAIN, CHUNK), jnp.int32),
          pltpu.VMEM((CH_MAIN, CHUNK), jnp.int32),
          pltpu.VMEM((TAIL,), jnp.int32),
          pltpu.VMEM((TAIL,), jnp.int32),
          pltpu.VMEM((CHUNK, D), jnp.float32),
          pltpu.VMEM((TAIL, D), jnp.float32),
          pltpu.VMEM_SHARED((N_NODES, D), jnp.float32),
          pltpu.SemaphoreType.DMA,
      ],
  )
  def k(feat_hbm, srcm_hbm, dstm_hbm, srct_hbm, dstt_hbm, zero_hbm, out_hbm,
        srcm_v, dstm_v, srct_v, dstt_v, rows_v, tail_v, acc_s, sem):
    c = lax.axis_index("c")
    s = lax.axis_index("s")
    wid = s * NC + c

    pltpu.sync_copy(zero_hbm,
                    acc_s.at[pl.ds(s * ROWS_PER_TILE, ROWS_PER_TILE)])

    @pl.when(s == NS - 1)
    def _():
      pltpu.sync_copy(zero_hbm.at[pl.ds(0, ROWS_REM)],
                      acc_s.at[pl.ds(NS * ROWS_PER_TILE, ROWS_REM)])

    pltpu.sync_copy(srcm_hbm.at[wid], srcm_v)
    pltpu.sync_copy(dstm_hbm.at[wid], dstm_v)
    pltpu.sync_copy(srct_hbm.at[pl.ds(wid * TAIL, TAIL)], srct_v)
    pltpu.sync_copy(dstt_hbm.at[pl.ds(wid * TAIL, TAIL)], dstt_v)
    plsc.subcore_barrier()

    def body(j, carry):
      pltpu.async_copy(feat_hbm.at[srcm_v.at[j]], rows_v, sem).wait()
      pltpu.sync_copy(rows_v, acc_s.at[dstm_v.at[j]], add=True)
      return carry

    lax.fori_loop(0, CH_MAIN, body, 0)

    pltpu.async_copy(feat_hbm.at[srct_v], tail_v, sem).wait()
    pltpu.sync_copy(tail_v, acc_s.at[dstt_v], add=True)

    plsc.subcore_barrier()
    pltpu.sync_copy(
        acc_s.at[pl.ds(s * ROWS_PER_TILE, ROWS_PER_TILE)],
        out_hbm.at[pl.ds(c * N_NODES + s * ROWS_PER_TILE, ROWS_PER_TILE)])

    @pl.when(s == NS - 1)
    def _():
      pltpu.sync_copy(
          acc_s.at[pl.ds(NS * ROWS_PER_TILE, ROWS_REM)],
          out_hbm.at[pl.ds(c * N_NODES + NS * ROWS_PER_TILE, ROWS_REM)])

  return k(feature, src_main, dst_main, src_tail, dst_tail, zeros)


def _tc_linear_kernel(h0_ref, h1_ref, w_ref, b_ref, o_ref):
  h = h0_ref[...] + h1_ref[...]
  o_ref[...] = lax.dot_general(
      h, w_ref[...], (((1,), (1,)), ((), ())),
      preferred_element_type=jnp.float32) + b_ref[...]


def kernel(feature, edge_index, W, b):
  src = edge_index[0]
  dst = edge_index[1]
  n_main = NW * CH_MAIN * CHUNK
  src_main = src[:n_main].reshape(NW, CH_MAIN, CHUNK)
  dst_main = dst[:n_main].reshape(NW, CH_MAIN, CHUNK)
  src_tail = src[n_main:]
  dst_tail = dst[n_main:]
  zeros = jnp.zeros((ROWS_PER_TILE, D), jnp.float32)

  partials = _sc_gather_scatter(feature, src_main, dst_main,
                                src_tail, dst_tail, zeros)

  out = pl.pallas_call(
      _tc_linear_kernel,
      grid=(1,),
      in_specs=[
          pl.BlockSpec((N_NODES, D), lambda i: (0, 0)),
          pl.BlockSpec((N_NODES, D), lambda i: (1, 0)),
          pl.BlockSpec((D, D), lambda i: (0, 0)),
          pl.BlockSpec((1, D), lambda i: (0, 0)),
      ],
      out_specs=pl.BlockSpec((N_NODES, D), lambda i: (0, 0)),
      out_shape=jax.ShapeDtypeStruct((N_NODES, D), jnp.float32),
  )(partials, partials, W, b.reshape(1, D))
  return out


# speedup vs baseline: 11.5678x; 1.3782x over previous
import functools

import jax
import jax.numpy as jnp
from jax import lax
from jax.experimental import pallas as pl
from jax.experimental.pallas import tpu as pltpu
from jax.experimental.pallas import tpu_sc as plsc

N_NODES = 10000
N_EDGES = 320000
D = 128

NC = 2
NS = 16
NW = NC * NS

CHUNK = 128
E_PER_W = N_EDGES // NW
CH = E_PER_W // CHUNK
TAIL = E_PER_W - CH * CHUNK
STAGES = 3
SLAB = CH // STAGES
ROWS_PER_TILE = 624
ROWS_REM = N_NODES - NS * ROWS_PER_TILE


def _sc_gather_scatter(feature, src_main, dst_main, src_tail, dst_tail, zeros):
  mesh = plsc.VectorSubcoreMesh(core_axis_name="c", subcore_axis_name="s")

  @functools.partial(
      pl.kernel,
      mesh=mesh,
      out_type=jax.ShapeDtypeStruct((NC * N_NODES, D), jnp.float32),
      scratch_types=[
          pltpu.VMEM((SLAB, CHUNK), jnp.int32),
          pltpu.VMEM((SLAB, CHUNK), jnp.int32),
          pltpu.VMEM((TAIL,), jnp.int32),
          pltpu.VMEM((TAIL,), jnp.int32),
          pltpu.VMEM((CHUNK, D), jnp.float32),
          pltpu.VMEM((CHUNK, D), jnp.float32),
          pltpu.VMEM((TAIL, D), jnp.float32),
          pltpu.VMEM_SHARED((N_NODES, D), jnp.float32),
          pltpu.SemaphoreType.DMA,
          pltpu.SemaphoreType.DMA,
      ],
  )
  def k(feat_hbm, srcm_hbm, dstm_hbm, srct_hbm, dstt_hbm, zero_hbm, out_hbm,
        src_v, dst_v, srct_v, dstt_v, rows0_v, rows1_v, tail_v, acc_s,
        sem0, sem1):
    c = lax.axis_index("c")
    s = lax.axis_index("s")
    wid = s * NC + c

    pltpu.sync_copy(zero_hbm,
                    acc_s.at[pl.ds(s * ROWS_PER_TILE, ROWS_PER_TILE)])

    @pl.when(s == NS - 1)
    def _():
      pltpu.sync_copy(zero_hbm.at[pl.ds(0, ROWS_REM)],
                      acc_s.at[pl.ds(NS * ROWS_PER_TILE, ROWS_REM)])

    pltpu.sync_copy(srct_hbm.at[pl.ds(wid * TAIL, TAIL)], srct_v)
    pltpu.sync_copy(dstt_hbm.at[pl.ds(wid * TAIL, TAIL)], dstt_v)
    plsc.subcore_barrier()

    bufs = (rows0_v, rows1_v)
    sems = (sem0, sem1)

    def start_gather(j, b):
      pltpu.async_copy(feat_hbm.at[src_v.at[j]], bufs[b], sems[b])

    def finish_chunk(j, b):
      pltpu.make_async_copy(feat_hbm.at[src_v.at[j]], bufs[b], sems[b]).wait()
      pltpu.sync_copy(bufs[b], acc_s.at[dst_v.at[j]], add=True)

    def stage_body(st, carry):
      pltpu.sync_copy(srcm_hbm.at[wid * STAGES + st], src_v)
      pltpu.sync_copy(dstm_hbm.at[wid * STAGES + st], dst_v)
      start_gather(0, 0)

      def body(i, carry2):
        j = 2 * i
        start_gather(j + 1, 1)
        finish_chunk(j, 0)

        @pl.when(j + 2 < SLAB)
        def _():
          start_gather(j + 2, 0)

        finish_chunk(j + 1, 1)
        return carry2

      lax.fori_loop(0, SLAB // 2, body, 0)
      return carry

    lax.fori_loop(0, STAGES, stage_body, 0)

    pltpu.async_copy(feat_hbm.at[srct_v], tail_v, sem0).wait()
    pltpu.sync_copy(tail_v, acc_s.at[dstt_v], add=True)

    plsc.subcore_barrier()
    pltpu.sync_copy(
        acc_s.at[pl.ds(s * ROWS_PER_TILE, ROWS_PER_TILE)],
        out_hbm.at[pl.ds(c * N_NODES + s * ROWS_PER_TILE, ROWS_PER_TILE)])

    @pl.when(s == NS - 1)
    def _():
      pltpu.sync_copy(
          acc_s.at[pl.ds(NS * ROWS_PER_TILE, ROWS_REM)],
          out_hbm.at[pl.ds(c * N_NODES + NS * ROWS_PER_TILE, ROWS_REM)])

  return k(feature, src_main, dst_main, src_tail, dst_tail, zeros)


def _tc_linear_kernel(h0_ref, h1_ref, w_ref, b_ref, o_ref):
  h = h0_ref[...] + h1_ref[...]
  o_ref[...] = lax.dot_general(
      h, w_ref[...], (((1,), (1,)), ((), ())),
      preferred_element_type=jnp.float32) + b_ref[...]


def kernel(feature, edge_index, W, b):
  src = edge_index[0]
  dst = edge_index[1]
  n_main = NW * CH * CHUNK
  src_main = src[:n_main].reshape(NW * STAGES, SLAB, CHUNK)
  dst_main = dst[:n_main].reshape(NW * STAGES, SLAB, CHUNK)
  src_tail = src[n_main:]
  dst_tail = dst[n_main:]
  zeros = jnp.zeros((ROWS_PER_TILE, D), jnp.float32)

  partials = _sc_gather_scatter(feature, src_main, dst_main,
                                src_tail, dst_tail, zeros)

  out = pl.pallas_call(
      _tc_linear_kernel,
      grid=(1,),
      in_specs=[
          pl.BlockSpec((N_NODES, D), lambda i: (0, 0)),
          pl.BlockSpec((N_NODES, D), lambda i: (1, 0)),
          pl.BlockSpec((D, D), lambda i: (0, 0)),
          pl.BlockSpec((1, D), lambda i: (0, 0)),
      ],
      out_specs=pl.BlockSpec((N_NODES, D), lambda i: (0, 0)),
      out_shape=jax.ShapeDtypeStruct((N_NODES, D), jnp.float32),
  )(partials, partials, W, b.reshape(1, D))
  return out
